# Initial kernel scaffold; baseline (speedup 1.0000x reference)
import jax
import jax.numpy as jnp
from jax.experimental import pallas as pl


def kernel(h, W, b, pad_h):
    raise NotImplementedError("write your pallas kernel here")


# speedup vs baseline: 5.5538x; 5.5538x over previous
import functools

import jax
import jax.numpy as jnp
from jax import lax
from jax.experimental import pallas as pl
from jax.experimental.pallas import tpu as pltpu
from jax.experimental.pallas import tpu_sc as plsc

_THRESHOLD = 0.99

_NC = 2
_NS = 16
_NW = _NC * _NS


def _iota16():
    return lax.iota(jnp.int32, 16)


def _cumsum16(kv):
    c = kv
    io = _iota16()
    for k in (1, 2, 4, 8):
        sh = c.at[jnp.maximum(io - k, 0)].get(mode="promise_in_bounds")
        c = c + jnp.where(io >= k, sh, 0)
    return c


def _mask_body(h_ref, w_ref, b_ref, keep_ref):
    hb = h_ref[0]
    w = w_ref[0]
    logit = jnp.sum(hb * w[None, :], axis=1) + b_ref[0]
    p = jax.nn.sigmoid(logit)
    keep_ref[0, 0, :] = (p < _THRESHOLD).astype(jnp.int32)


def _make_sc_pack(B, M, H):
    WPB = _NW // B
    TPW = M // WPB
    CH = 64
    NCH = TPW // CH

    mesh = plsc.VectorSubcoreMesh(core_axis_name="c", subcore_axis_name="s")

    @functools.partial(
        pl.kernel,
        mesh=mesh,
        out_type=jax.ShapeDtypeStruct((B * M, H), jnp.float32),
        scratch_types=[
            pltpu.VMEM((M,), jnp.int32),
            pltpu.VMEM((NCH, CH), jnp.int32),
            pltpu.VMEM((CH, H), jnp.float32),
            pltpu.VMEM((H,), jnp.float32),
            pltpu.SMEM((NCH,), jnp.int32),
            pltpu.SemaphoreType.DMA,
        ],
    )
    def sc_pack(h_hbm, keep_hbm, pad_hbm, out_hbm, keep_v, dest_v, buf_v,
                pad_v, ccnt_s, sem):
        cid = lax.axis_index("c")
        sid = lax.axis_index("s")
        wid = sid * _NC + cid
        b_id = wid // WPB
        q = wid % WPB
        base = b_id * M
        tbase = q * TPW

        pltpu.sync_copy(keep_hbm.at[pl.ds(base, M)], keep_v)
        pltpu.sync_copy(pad_hbm.at[0], pad_v)

        def _acc(j, kt):
            return kt + _cumsum16(keep_v[pl.ds(j * 16, 16)])[15]
        kt_pre = lax.fori_loop(0, tbase // 16, _acc, jnp.int32(0))
        ex_pre = tbase - kt_pre
        count_b = lax.fori_loop(0, M // 16, _acc, jnp.int32(0))

        def _dest_chunk(k, carry):
            kt, ex = carry
            s_chunk = jnp.int32(0)
            for g in range(CH // 16):
                off = tbase + k * CH + g * 16
                kv = keep_v[pl.ds(off, 16)]
                c_inc = _cumsum16(kv)
                e_inc = (_iota16() + 1) - c_inc
                dest = jnp.where(kv > 0,
                                 kt + c_inc - 1,
                                 count_b + ex + e_inc - 1) + base
                dest_v[k, pl.ds(g * 16, 16)] = dest
                csum = c_inc[15]
                s_chunk = s_chunk + csum
                kt = kt + csum
                ex = ex + (16 - csum)
            ccnt_s[k] = s_chunk
            return kt, ex
        lax.fori_loop(0, NCH, _dest_chunk, (kt_pre, ex_pre))

        def _move_chunk(k, _):
            row0 = base + tbase + k * CH
            pltpu.sync_copy(h_hbm.at[pl.ds(row0, CH)], buf_v)
            koff = tbase + k * CH

            @pl.when(ccnt_s[k] < CH)
            def _patch():
                def _row(r, _c):
                    kv = keep_v[pl.ds(koff + (r // 16) * 16, 16)]
                    flag = _cumsum16(jnp.where(_iota16() == r % 16, kv, 0))[15]

                    @pl.when(flag == 0)
                    def _z():
                        def _col(c, _d):
                            buf_v[r, pl.ds(c * 16, 16)] = pad_v[pl.ds(c * 16, 16)]
                            return _d
                        lax.fori_loop(0, H // 16, _col, 0)
                    return _c
                lax.fori_loop(0, CH, _row, 0)

            pltpu.async_copy(buf_v, out_hbm.at[dest_v.at[k]], sem).wait()
            return 0
        lax.fori_loop(0, NCH, _move_chunk, 0)

    return sc_pack


def kernel(h, W, b, pad_h):
    B, M, H = h.shape

    keep = pl.pallas_call(
        _mask_body,
        grid=(B,),
        in_specs=[
            pl.BlockSpec((1, M, H), lambda i: (i, 0, 0)),
            pl.BlockSpec((1, H), lambda i: (0, 0)),
            pl.BlockSpec(memory_space=pltpu.SMEM),
        ],
        out_specs=pl.BlockSpec((1, 1, M), lambda i: (i, 0, 0)),
        out_shape=jax.ShapeDtypeStruct((B, 1, M), jnp.int32),
    )(h, W, b)

    sc_pack = _make_sc_pack(B, M, H)
    out = sc_pack(h.reshape(B * M, H), keep.reshape(B * M), pad_h)
    return out.reshape(B, M, H)


# speedup vs baseline: 6.3679x; 1.1466x over previous
import functools

import jax
import jax.numpy as jnp
from jax import lax
from jax.experimental import pallas as pl
from jax.experimental.pallas import tpu as pltpu
from jax.experimental.pallas import tpu_sc as plsc

_THRESHOLD = 0.99

_NC = 2
_NS = 16
_NW = _NC * _NS


def _iota16():
    return lax.iota(jnp.int32, 16)


def _cumsum16(kv):
    c = kv
    io = _iota16()
    for k in (1, 2, 4, 8):
        sh = c.at[jnp.maximum(io - k, 0)].get(mode="promise_in_bounds")
        c = c + jnp.where(io >= k, sh, 0)
    return c


def _mask_body(h_ref, w_ref, b_ref, keep_ref, cnt_ref):
    hb = h_ref[0]
    w = w_ref[0]
    wmat = jnp.broadcast_to(w[:, None], (w.shape[0], 128))
    logit = jnp.dot(hb, wmat,
                    preferred_element_type=jnp.float32)[:, 0] + b_ref[0]
    p = jax.nn.sigmoid(logit)
    ki = (p < _THRESHOLD).astype(jnp.int32)
    keep_ref[0, 0, :] = ki
    cnt_ref[0, 0, :] = jnp.broadcast_to(jnp.sum(ki), (128,))


def _make_sc_pack(B, M, H):
    WPB = _NW // B
    TPW = M // WPB
    CH = 64
    NCH = TPW // CH

    mesh = plsc.VectorSubcoreMesh(core_axis_name="c", subcore_axis_name="s")

    @functools.partial(
        pl.kernel,
        mesh=mesh,
        out_type=jax.ShapeDtypeStruct((B * M, H), jnp.float32),
        scratch_types=[
            pltpu.VMEM((M,), jnp.int32),
            pltpu.VMEM((NCH, CH), jnp.int32),
            pltpu.VMEM((2, CH, H), jnp.float32),
            pltpu.VMEM((H,), jnp.float32),
            pltpu.VMEM((16,), jnp.int32),
            pltpu.SMEM((NCH,), jnp.int32),
            pltpu.SemaphoreType.DMA,
            pltpu.SemaphoreType.DMA,
        ],
    )
    def sc_pack(h_hbm, keep_hbm, cnt_hbm, pad_hbm, out_hbm, keep_v, dest_v,
                buf_v, pad_v, cnt_v, ccnt_s, rsem, wsem):
        cid = lax.axis_index("c")
        sid = lax.axis_index("s")
        wid = sid * _NC + cid
        b_id = wid // WPB
        q = wid % WPB
        base = b_id * M
        tbase = q * TPW
        rbase = base + tbase

        pltpu.sync_copy(cnt_hbm.at[pl.ds(b_id * 128, 16)], cnt_v)
        count_b = _cumsum16(jnp.where(_iota16() == 0, cnt_v[pl.ds(0, 16)],
                                      0))[15]

        def _general_path():
            pltpu.sync_copy(keep_hbm.at[pl.ds(base, M)], keep_v)
            pltpu.sync_copy(pad_hbm.at[0], pad_v)

            def _acc(j, kt):
                return kt + _cumsum16(keep_v[pl.ds(j * 16, 16)])[15]
            kt_pre = lax.fori_loop(0, tbase // 16, _acc, jnp.int32(0))
            ex_pre = tbase - kt_pre

            def _dest_chunk(k, carry):
                kt, ex = carry
                s_chunk = jnp.int32(0)
                for g in range(CH // 16):
                    off = tbase + k * CH + g * 16
                    kv = keep_v[pl.ds(off, 16)]
                    c_inc = _cumsum16(kv)
                    e_inc = (_iota16() + 1) - c_inc
                    dest = jnp.where(kv > 0,
                                     kt + c_inc - 1,
                                     count_b + ex + e_inc - 1) + base
                    dest_v[k, pl.ds(g * 16, 16)] = dest
                    csum = c_inc[15]
                    s_chunk = s_chunk + csum
                    kt = kt + csum
                    ex = ex + (16 - csum)
                ccnt_s[k] = s_chunk
                return kt, ex
            lax.fori_loop(0, NCH, _dest_chunk, (kt_pre, ex_pre))

            pltpu.async_copy(h_hbm.at[pl.ds(rbase, CH)], buf_v.at[0], rsem)

            def _move_chunk(k, _):
                slot = k % 2
                nslot = (k + 1) % 2
                koff = tbase + k * CH

                @pl.when(k > 0)
                def _():
                    pltpu.make_async_copy(
                        buf_v.at[nslot], out_hbm.at[dest_v.at[k - 1]],
                        wsem).wait()

                @pl.when(k + 1 < NCH)
                def _():
                    pltpu.async_copy(
                        h_hbm.at[pl.ds(rbase + (k + 1) * CH, CH)],
                        buf_v.at[nslot], rsem)

                pltpu.make_async_copy(
                    h_hbm.at[pl.ds(rbase, CH)], buf_v.at[slot], rsem).wait()

                @pl.when(ccnt_s[k] < CH)
                def _patch():
                    def _row(r, _c):
                        kv = keep_v[pl.ds(koff + (r // 16) * 16, 16)]
                        flag = _cumsum16(
                            jnp.where(_iota16() == r % 16, kv, 0))[15]

                        @pl.when(flag == 0)
                        def _z():
                            def _col(c, _d):
                                buf_v[slot, r, pl.ds(c * 16, 16)] = (
                                    pad_v[pl.ds(c * 16, 16)])
                                return _d
                            lax.fori_loop(0, H // 16, _col, 0)
                        return _c
                    lax.fori_loop(0, CH, _row, 0)

                pltpu.async_copy(buf_v.at[slot], out_hbm.at[dest_v.at[k]],
                                 wsem)
                return 0
            lax.fori_loop(0, NCH, _move_chunk, 0)
            pltpu.make_async_copy(
                buf_v.at[(NCH - 1) % 2], out_hbm.at[dest_v.at[NCH - 1]],
                wsem).wait()

        @pl.when(count_b == M)
        def _identity():
            pltpu.async_copy(h_hbm.at[pl.ds(rbase, CH)], buf_v.at[0], rsem)

            def _lin(k, _):
                slot = k % 2
                nslot = (k + 1) % 2

                @pl.when(k > 0)
                def _():
                    pltpu.make_async_copy(
                        buf_v.at[nslot],
                        out_hbm.at[pl.ds(rbase + (k - 1) * CH, CH)],
                        wsem).wait()

                @pl.when(k + 1 < NCH)
                def _():
                    pltpu.async_copy(
                        h_hbm.at[pl.ds(rbase + (k + 1) * CH, CH)],
                        buf_v.at[nslot], rsem)

                pltpu.make_async_copy(
                    h_hbm.at[pl.ds(rbase, CH)], buf_v.at[slot], rsem).wait()
                pltpu.async_copy(buf_v.at[slot],
                                 out_hbm.at[pl.ds(rbase + k * CH, CH)], wsem)
                return 0
            lax.fori_loop(0, NCH, _lin, 0)
            pltpu.make_async_copy(
                buf_v.at[(NCH - 1) % 2],
                out_hbm.at[pl.ds(rbase + (NCH - 1) * CH, CH)], wsem).wait()

        @pl.when(count_b < M)
        def _general():
            _general_path()

    return sc_pack


def kernel(h, W, b, pad_h):
    B, M, H = h.shape

    keep, counts = pl.pallas_call(
        _mask_body,
        grid=(B,),
        in_specs=[
            pl.BlockSpec((1, M, H), lambda i: (i, 0, 0)),
            pl.BlockSpec((1, H), lambda i: (0, 0)),
            pl.BlockSpec(memory_space=pltpu.SMEM),
        ],
        out_specs=[
            pl.BlockSpec((1, 1, M), lambda i: (i, 0, 0)),
            pl.BlockSpec((1, 1, 128), lambda i: (i, 0, 0)),
        ],
        out_shape=[
            jax.ShapeDtypeStruct((B, 1, M), jnp.int32),
            jax.ShapeDtypeStruct((B, 1, 128), jnp.int32),
        ],
    )(h, W, b)

    sc_pack = _make_sc_pack(B, M, H)
    out = sc_pack(h.reshape(B * M, H), keep.reshape(B * M),
                  counts.reshape(B * 128), pad_h)
    return out.reshape(B, M, H)
